# baseline (speedup 1.0000x reference)
import jax
import jax.numpy as jnp
from jax.experimental import pallas as pl


_BLK = 3584


def _merge_blk(xs_ref, w_ref, b_ref, emb_ref, s_ref):
    d = xs_ref.shape[-1]
    blk = xs_ref.shape[1]
    b = b_ref[0]

    q = xs_ref[0]
    m1 = xs_ref[1]
    m2 = xs_ref[2]
    m3 = xs_ref[3]

    zcol = jnp.zeros((d, 1), dtype=q.dtype)
    ocol = jnp.ones((d, 1), dtype=q.dtype)
    ones_lo = jnp.concatenate([zcol, ocol], axis=0)
    r_msg = jnp.concatenate(
        [jnp.concatenate([w_ref[:d, :], zcol], axis=0), ones_lo], axis=1
    )
    r_q = jnp.concatenate(
        [jnp.concatenate([w_ref[d:, :], zcol], axis=0), ones_lo], axis=1
    )

    def red(x, r):
        aug = jnp.concatenate([x, x * x], axis=1)
        out = jnp.dot(aug, r, preferred_element_type=jnp.float32)
        out_t = out.T
        return out_t[1:2, :], out_t[0:1, :]

    ssq_q, dot_q = red(q, r_q)
    ssq_1, dot_1 = red(m1, r_msg)
    ssq_2, dot_2 = red(m2, r_msg)
    ssq_3, dot_3 = red(m3, r_msg)

    def inv_norm(ssq):
        return jax.lax.rsqrt(jnp.maximum(ssq, 1e-24))

    iq = inv_norm(ssq_q)
    i1 = inv_norm(ssq_1)
    i2 = inv_norm(ssq_2)
    i3 = inv_norm(ssq_3)

    qterm = dot_q * iq + b

    def score(dt, inv):
        s = dt * inv + qterm
        return jnp.where(s >= 0, s, 0.01 * s)

    s1 = score(dot_1, i1)
    s2 = score(dot_2, i2)
    s3 = score(dot_3, i3)
    smax = jnp.maximum(jnp.maximum(s1, s2), s3)
    e1 = jnp.exp(s1 - smax)
    e2 = jnp.exp(s2 - smax)
    e3 = jnp.exp(s3 - smax)
    r = 1.0 / (e1 + e2 + e3 + 1e-16)
    a1 = e1 * r
    a2 = e2 * r
    a3 = e3 * r

    s_ref[0:1, :] = a1
    s_ref[1:2, :] = a2
    s_ref[2:3, :] = a3

    cm = jnp.concatenate([iq, a1 * i1, a2 * i2, a3 * i3], axis=0)
    ct = cm.T
    lane = jax.lax.broadcasted_iota(jnp.int32, (4, 4 * d), 1) // d
    row = jax.lax.broadcasted_iota(jnp.int32, (4, 4 * d), 0)
    one_map = (lane == row).astype(q.dtype)
    bc = jnp.dot(ct, one_map, preferred_element_type=jnp.float32)
    emb_ref[...] = (
        bc[:, 0:d] * q
        + bc[:, d : 2 * d] * m1
        + bc[:, 2 * d : 3 * d] * m2
        + bc[:, 3 * d : 4 * d] * m3
    )


def kernel(xs, W_att, b_att):
    L, N, d = xs.shape
    blk = _BLK
    grid = ((N + blk - 1) // blk,)
    emb, sc = pl.pallas_call(
        _merge_blk,
        grid=grid,
        in_specs=[
            pl.BlockSpec((L, blk, d), lambda i: (0, i, 0)),
            pl.BlockSpec((2 * d, 1), lambda i: (0, 0)),
            pl.BlockSpec((1,), lambda i: (0,)),
        ],
        out_specs=[
            pl.BlockSpec((blk, d), lambda i: (i, 0)),
            pl.BlockSpec((L - 1, blk), lambda i: (0, i)),
        ],
        out_shape=[
            jax.ShapeDtypeStruct((N, d), xs.dtype),
            jax.ShapeDtypeStruct((L - 1, N), xs.dtype),
        ],
    )(xs, W_att, b_att)
    return emb, sc.reshape(-1)


# speedup vs baseline: 1.0276x; 1.0276x over previous
import jax
import jax.numpy as jnp
from jax.experimental import pallas as pl
from jax.experimental.pallas import tpu as pltpu


_BLK = 6272


def _merge_blk(xs_ref, w_ref, b_ref, emb_ref, s_ref):
    d = xs_ref.shape[-1]
    blk = xs_ref.shape[1]
    b = b_ref[0]

    q = xs_ref[0]
    m1 = xs_ref[1]
    m2 = xs_ref[2]
    m3 = xs_ref[3]

    zcol = jnp.zeros((d, 1), dtype=q.dtype)
    ocol = jnp.ones((d, 1), dtype=q.dtype)
    ones_lo = jnp.concatenate([zcol, ocol], axis=0)
    r_msg = jnp.concatenate(
        [jnp.concatenate([w_ref[:d, :], zcol], axis=0), ones_lo], axis=1
    )
    r_q = jnp.concatenate(
        [jnp.concatenate([w_ref[d:, :], zcol], axis=0), ones_lo], axis=1
    )

    def red(x, r):
        aug = jnp.concatenate([x, x * x], axis=1)
        out = jnp.dot(aug, r, preferred_element_type=jnp.float32)
        out_t = out.T
        return out_t[1:2, :], out_t[0:1, :]

    ssq_q, dot_q = red(q, r_q)
    ssq_1, dot_1 = red(m1, r_msg)
    ssq_2, dot_2 = red(m2, r_msg)
    ssq_3, dot_3 = red(m3, r_msg)

    def inv_norm(ssq):
        return jax.lax.rsqrt(jnp.maximum(ssq, 1e-24))

    iq = inv_norm(ssq_q)
    i1 = inv_norm(ssq_1)
    i2 = inv_norm(ssq_2)
    i3 = inv_norm(ssq_3)

    qterm = dot_q * iq + b

    def score(dt, inv):
        s = dt * inv + qterm
        return jnp.where(s >= 0, s, 0.01 * s)

    s1 = score(dot_1, i1)
    s2 = score(dot_2, i2)
    s3 = score(dot_3, i3)
    smax = jnp.maximum(jnp.maximum(s1, s2), s3)
    e1 = jnp.exp(s1 - smax)
    e2 = jnp.exp(s2 - smax)
    e3 = jnp.exp(s3 - smax)
    r = 1.0 / (e1 + e2 + e3 + 1e-16)
    a1 = e1 * r
    a2 = e2 * r
    a3 = e3 * r

    s_ref[0:1, :] = a1
    s_ref[1:2, :] = a2
    s_ref[2:3, :] = a3

    cm = jnp.concatenate([iq, a1 * i1, a2 * i2, a3 * i3], axis=0)
    ct = cm.T
    lane = jax.lax.broadcasted_iota(jnp.int32, (4, 4 * d), 1) // d
    row = jax.lax.broadcasted_iota(jnp.int32, (4, 4 * d), 0)
    one_map = (lane == row).astype(q.dtype)
    bc = jnp.dot(ct, one_map, preferred_element_type=jnp.float32)
    emb_ref[...] = (
        bc[:, 0:d] * q
        + bc[:, d : 2 * d] * m1
        + bc[:, 2 * d : 3 * d] * m2
        + bc[:, 3 * d : 4 * d] * m3
    )


def kernel(xs, W_att, b_att):
    L, N, d = xs.shape
    blk = _BLK
    grid = ((N + blk - 1) // blk,)
    emb, sc = pl.pallas_call(
        _merge_blk,
        grid=grid,
        in_specs=[
            pl.BlockSpec((L, blk, d), lambda i: (0, i, 0)),
            pl.BlockSpec((2 * d, 1), lambda i: (0, 0)),
            pl.BlockSpec((1,), lambda i: (0,)),
        ],
        out_specs=[
            pl.BlockSpec((blk, d), lambda i: (i, 0)),
            pl.BlockSpec((L - 1, blk), lambda i: (0, i)),
        ],
        out_shape=[
            jax.ShapeDtypeStruct((N, d), xs.dtype),
            jax.ShapeDtypeStruct((L - 1, N), xs.dtype),
        ],
        compiler_params=pltpu.CompilerParams(
            dimension_semantics=("parallel",),
        ),
    )(xs, W_att, b_att)
    return emb, sc.reshape(-1)
